# baseline (speedup 1.0000x reference)
import functools

import jax
import jax.numpy as jnp
from jax import lax
from jax.experimental import pallas as pl
from jax.experimental.pallas import tpu as pltpu
from jax.experimental.pallas import tpu_sc as plsc

_BATCH = 16384
_EMB = 16
_OUT = 10
_NIG = 10
_NGG = 21

_info = plsc.get_sparse_core_info()
_NC = _info.num_cores
_NS = _info.num_subcores
_NW = _NC * _NS
_BPW = _BATCH // _NW

_sc_mesh = plsc.VectorSubcoreMesh(core_axis_name="c", subcore_axis_name="s")


@functools.partial(
    pl.kernel,
    mesh=_sc_mesh,
    out_type=jax.ShapeDtypeStruct((_BATCH, _EMB), jnp.float32),
    scratch_types=[
        pltpu.VMEM((_BPW,), jnp.int32),
        pltpu.VMEM((_BPW, _EMB), jnp.float32),
        pltpu.SemaphoreType.DMA,
    ],
    compiler_params=pltpu.CompilerParams(use_tc_tiling_on_sc=False),
)
def _sc_gather(table_hbm, idx_hbm, out_hbm, idx_v, rows_v, sem):
    wid = lax.axis_index("s") * _NC + lax.axis_index("c")
    base = wid * _BPW
    pltpu.sync_copy(idx_hbm.at[pl.ds(base, _BPW)], idx_v)
    pltpu.async_copy(table_hbm.at[idx_v], rows_v, sem).wait()
    pltpu.sync_copy(rows_v, out_hbm.at[pl.ds(base, _BPW)])


_ROWS = 2048


def _tc_body(g_ref, ig_ref, gg_ref, w0_ref, wig_ref, wgg_ref, b_ref, o_ref):
    g = jnp.maximum(g_ref[...], 0.0)
    acc = jnp.dot(g, w0_ref[...], preferred_element_type=jnp.float32)
    oh_ig = (lax.broadcasted_iota(jnp.int32, (_ROWS, _NIG), 1)
             == ig_ref[...]).astype(jnp.float32)
    acc += jnp.dot(oh_ig, wig_ref[...], preferred_element_type=jnp.float32)
    oh_gg = (lax.broadcasted_iota(jnp.int32, (_ROWS, _NGG), 1)
             == gg_ref[...]).astype(jnp.float32)
    acc += jnp.dot(oh_gg, wgg_ref[...], preferred_element_type=jnp.float32)
    o_ref[...] = acc + b_ref[...]


def _tc_dense(g, ig2, gg2, w0, wig, wgg, b2):
    grid = _BATCH // _ROWS
    return pl.pallas_call(
        _tc_body,
        grid=(grid,),
        in_specs=[
            pl.BlockSpec((_ROWS, _EMB), lambda i: (i, 0)),
            pl.BlockSpec((_ROWS, 1), lambda i: (i, 0)),
            pl.BlockSpec((_ROWS, 1), lambda i: (i, 0)),
            pl.BlockSpec((_EMB, _OUT), lambda i: (0, 0)),
            pl.BlockSpec((_NIG, _OUT), lambda i: (0, 0)),
            pl.BlockSpec((_NGG, _OUT), lambda i: (0, 0)),
            pl.BlockSpec((1, _OUT), lambda i: (0, 0)),
        ],
        out_specs=pl.BlockSpec((_ROWS, _OUT), lambda i: (i, 0)),
        out_shape=jax.ShapeDtypeStruct((_BATCH, _OUT), jnp.float32),
    )(g, ig2, gg2, w0, wig, wgg, b2)


@jax.jit
def kernel(item_indices, index_group_indices, garment_group_indices, emb_table, W, b):
    item = item_indices.astype(jnp.int32)
    ig2 = index_group_indices.astype(jnp.int32).reshape(_BATCH, 1)
    gg2 = garment_group_indices.astype(jnp.int32).reshape(_BATCH, 1)
    g = _sc_gather(emb_table, item)
    w0 = W[:_EMB]
    wig = W[_EMB:_EMB + _NIG]
    wgg = W[_EMB + _NIG:]
    return _tc_dense(g, ig2, gg2, w0, wig, wgg, b.reshape(1, _OUT))
